# Initial kernel scaffold; baseline (speedup 1.0000x reference)
import jax
import jax.numpy as jnp
from jax.experimental import pallas as pl


def kernel(x, W, b):
    raise NotImplementedError("write your pallas kernel here")


# speedup vs baseline: 2.4756x; 2.4756x over previous
import functools

import jax
import jax.numpy as jnp
from jax.experimental import pallas as pl
from jax.experimental.pallas import tpu as pltpu

NUM_TOKENS = 16384
INPUT_DIM = 2048
NUM_EXPERTS = 64
NOISE_STD = 1.0
BLOCK_T = 1024


def _gating_body(x_ref, w_ref, b_ref, n_ref, o_ref):
    logits = jax.lax.dot_general(
        x_ref[...], w_ref[...],
        dimension_numbers=(((1,), (1,)), ((), ())),
        preferred_element_type=jnp.float32,
    )
    noisy = logits + b_ref[...] + n_ref[...]

    col = jax.lax.broadcasted_iota(jnp.int32, noisy.shape, 1)
    m1 = jnp.max(noisy, axis=-1, keepdims=True)
    i1 = jnp.min(jnp.where(noisy == m1, col, NUM_EXPERTS), axis=-1,
                 keepdims=True)
    is1 = col == i1
    masked = jnp.where(is1, -jnp.inf, noisy)
    m2 = jnp.max(masked, axis=-1, keepdims=True)
    i2 = jnp.min(jnp.where(masked == m2, col, NUM_EXPERTS), axis=-1,
                 keepdims=True)
    is2 = col == i2

    t = jnp.exp(m2 - m1)
    p1 = 1.0 / (1.0 + t)
    o_ref[...] = jnp.where(is1, p1, 0.0) + jnp.where(is2, t * p1, 0.0)


@jax.jit
def kernel(x, W, b):
    n_tokens = x.shape[0]
    noise = jax.random.normal(jax.random.key(1), (n_tokens, NUM_EXPERTS),
                              dtype=jnp.float32) * NOISE_STD
    grid = (n_tokens // BLOCK_T,)
    return pl.pallas_call(
        _gating_body,
        grid=grid,
        in_specs=[
            pl.BlockSpec((BLOCK_T, INPUT_DIM), lambda i: (i, 0)),
            pl.BlockSpec((NUM_EXPERTS, INPUT_DIM), lambda i: (0, 0)),
            pl.BlockSpec((1, NUM_EXPERTS), lambda i: (0, 0)),
            pl.BlockSpec((BLOCK_T, NUM_EXPERTS), lambda i: (i, 0)),
        ],
        out_specs=pl.BlockSpec((BLOCK_T, NUM_EXPERTS), lambda i: (i, 0)),
        out_shape=jax.ShapeDtypeStruct((n_tokens, NUM_EXPERTS), jnp.float32),
    )(x, W, b.reshape(1, NUM_EXPERTS), noise)
